# baseline (speedup 1.0000x reference)
import functools

import jax
import jax.numpy as jnp
from jax import lax
from jax.experimental import pallas as pl
from jax.experimental.pallas import tpu as pltpu
from jax.experimental.pallas import tpu_sc as plsc

NUM_EMB = 1000000
DIM = 64
B_TOTAL = 16384 * 50

_info = plsc.get_sparse_core_info()
NC, NS = _info.num_cores, _info.num_subcores
NW = NC * NS
BPW = B_TOTAL // NW
CHUNK = 128
NCHUNK = BPW // CHUNK
GROUP = 4
NGROUP = NCHUNK // GROUP

_mesh = plsc.VectorSubcoreMesh(core_axis_name="c", subcore_axis_name="s")


@functools.partial(
    pl.kernel,
    mesh=_mesh,
    out_type=jax.ShapeDtypeStruct((NW * NGROUP, GROUP, CHUNK, DIM), jnp.float32),
    scratch_types=[
        pltpu.VMEM((NCHUNK, CHUNK), jnp.int32),
        pltpu.VMEM((2, GROUP, CHUNK, DIM), jnp.float32),
        pltpu.SemaphoreType.DMA((2,)),
        pltpu.SemaphoreType.DMA((2,)),
    ],
    compiler_params=pltpu.CompilerParams(use_tc_tiling_on_sc=False),
)
def _emb_lookup(table_hbm, idx_hbm, out_hbm, idx_all, rows, gsem, ssem):
    wid = lax.axis_index("s") * NC + lax.axis_index("c")
    base_chunk = wid * NCHUNK
    base_group = wid * NGROUP

    pltpu.sync_copy(idx_hbm.at[pl.ds(base_chunk, NCHUNK)], idx_all)

    def fire_gathers(t, d):
        for b in range(GROUP):
            pltpu.async_copy(table_hbm.at[idx_all.at[t * GROUP + b]],
                             rows.at[d, b], gsem.at[d])

    def drain_gathers(t, d):
        for b in range(GROUP):
            pltpu.make_async_copy(table_hbm.at[idx_all.at[t * GROUP + b]],
                                  rows.at[d, b], gsem.at[d]).wait()

    def store_sized(t, d):
        return pltpu.make_async_copy(rows.at[d], out_hbm.at[base_group + t],
                                     ssem.at[d])

    fire_gathers(0, 0)

    def group_body(t, carry):
        d = lax.rem(t, 2)
        drain_gathers(t, d)

        @pl.when(t >= 1)
        def _():
            store_sized(t - 1, 1 - d).wait()

        @pl.when(t + 1 < NGROUP)
        def _():
            fire_gathers(t + 1, 1 - d)

        pltpu.async_copy(rows.at[d], out_hbm.at[base_group + t], ssem.at[d])
        return carry

    lax.fori_loop(0, NGROUP, group_body, 0)
    store_sized(NGROUP - 1, (NGROUP - 1) % 2).wait()


def kernel(x, weight):
    x_flat = x.reshape(NW * NCHUNK, CHUNK).astype(jnp.int32)
    out = _emb_lookup(weight, x_flat)
    return out.reshape(x.shape + (DIM,))
